# baseline (speedup 1.0000x reference)
import functools

import jax
import jax.numpy as jnp
from jax import lax
from jax.experimental import pallas as pl
from jax.experimental.pallas import tpu as pltpu
from jax.experimental.pallas import tpu_sc as plsc

_NUM_CLASSES = 1000000
_DROPOUT_PROB = 0.1
_NC = 2
_NS = 16
_NW = _NC * _NS
_L = 16
_SR = 128


def _iota16():
    return lax.iota(jnp.int32, _L)


def _splat(v):
    return jnp.full((_L,), v, jnp.int32)


@functools.lru_cache(maxsize=None)
def _make_lookup(vocab: int, d: int, b: int):
    n_slabs = d // 8
    n_tc = vocab // 128
    tail_owner = n_tc % _NW
    n_segs = 8
    mesh = plsc.VectorSubcoreMesh(core_axis_name="c", subcore_axis_name="s")

    @functools.partial(
        pl.kernel,
        out_type=jax.ShapeDtypeStruct((b + 8, 128), jnp.float32),
        mesh=mesh,
        scratch_types=[
            pltpu.VMEM((b,), jnp.int32),
            pltpu.VMEM((b,), jnp.int32),
            pltpu.VMEM((b,), jnp.int32),
            pltpu.VMEM((b,), jnp.int32),
            pltpu.VMEM((2, n_slabs, 8, 128), jnp.float32),
            pltpu.VMEM((2, _SR, 128), jnp.float32),
            pltpu.VMEM((2, _SR), jnp.int32),
            pltpu.SemaphoreType.DMA((2,)),
            pltpu.SemaphoreType.DMA,
        ],
        compiler_params=pltpu.CompilerParams(needs_layout_passes=False),
    )
    def lookup_kernel(idx_hbm, tab_t_hbm, tail_t_hbm, out_hbm,
                      idx_v, mlist, mlist2, clist, bufs, stage, brow,
                      semd, sems):
        wid = lax.axis_index("s") * _NC + lax.axis_index("c")
        n_own = jnp.where(wid < (n_tc % _NW),
                          (n_tc + _NW - 1) // _NW, n_tc // _NW)
        pltpu.sync_copy(idx_hbm, idx_v)
        for sl in range(2):
            for i in range(_SR // _L):
                plsc.store_scatter(brow, [_splat(sl), _iota16() + i * _L],
                                   _splat(b))

        def p1(v, nw):
            i_vec = idx_v[pl.ds(v * _L, _L)]
            m = ((i_vec >> 7) & (_NW - 1)) == wid
            plsc.store_compressed(mlist.at[pl.ds(nw, _L)],
                                  _iota16() + v * _L, mask=m)
            return nw + jnp.sum(m.astype(jnp.int32))

        nw = lax.fori_loop(0, b // _L, p1, 0, unroll=False)
        ngw = (nw + _L - 1) >> 4

        seg_bounds = [0]
        off = 0
        for seg in range(n_segs):
            def p2(g, o, _seg=seg):
                pos = _iota16() + g * _L
                valid = pos < nw
                b_vec = mlist[pl.ds(g * _L, _L)]
                bf = jnp.where(valid, b_vec, 0)
                i_vec = plsc.load_gather(idx_v, [bf])
                sm = jnp.logical_and(
                    valid, (((i_vec >> 7) - wid) >> 10) == _seg)
                plsc.store_compressed(mlist2.at[pl.ds(o, _L)], b_vec, mask=sm)
                return o + jnp.sum(sm.astype(jnp.int32))

            off = lax.fori_loop(0, ngw, p2, off, unroll=False)
            seg_bounds.append(off)

        def _fire(slot):
            pltpu.async_copy(stage.at[slot], out_hbm.at[brow.at[slot]], sems)

        def _drain_one():
            pltpu.make_async_copy(
                stage.at[0], out_hbm.at[brow.at[0]], sems).wait()

        def _flush(carry):
            def do(c):
                f, s, n = c
                _fire(s)
                ns = 1 - s
                lax.cond(n >= 1, _drain_one, lambda: None)
                for i in range(_SR // _L):
                    plsc.store_scatter(brow,
                                       [_splat(ns), _iota16() + i * _L],
                                       _splat(b))
                return (0, ns, n + 1)

            fill, _, _ = carry
            return lax.cond(fill >= _SR - _L, do, lambda c: c, carry)

        def rescan(cid, lo, hi):
            def gb(g, nc):
                off_g = lo + g * _L
                pos = _iota16() + off_g
                valid = pos < hi
                b_vec = mlist2[pl.ds(off_g, _L)]
                bf = jnp.where(valid, b_vec, 0)
                i_vec = plsc.load_gather(idx_v, [bf])
                cm = jnp.logical_and(valid, (i_vec >> 7) == cid)
                plsc.store_compressed(clist.at[pl.ds(nc, _L)], b_vec, mask=cm)
                return nc + jnp.sum(cm.astype(jnp.int32))

            return lax.fori_loop(0, (hi - lo + _L - 1) >> 4, gb, 0,
                                 unroll=False)

        def extract_groups(nc, c0, dslot, carry):
            def eb(e, car):
                fill, slot, nfired = _flush(car)
                pos = _iota16() + e * _L
                valid = pos < nc
                b_vec = clist[pl.ds(e * _L, _L)]
                bf = jnp.where(valid, b_vec, 0)
                i_vec = plsc.load_gather(idx_v, [bf])
                l_vec = jnp.where(valid, i_vec - c0, 0)
                sl = _splat(slot)
                dsl = _splat(dslot)
                fvec = _splat(fill) + _iota16()
                for k in range(n_slabs):
                    kv = _splat(k)
                    for s in range(8):
                        vals = plsc.load_gather(
                            bufs, [dsl, kv, _splat(s), l_vec])
                        plsc.store_scatter(
                            stage, [sl, fvec, _splat(8 * k + s)], vals)
                plsc.store_scatter(brow, [sl, fvec],
                                   jnp.where(valid, b_vec, b))
                return (fill + _L, slot, nfired)

            return lax.fori_loop(0, (nc + _L - 1) >> 4, eb, carry,
                                 unroll=False)

        def fire_column(t, slot):
            cid = wid + _NW * t
            c0 = pl.multiple_of(cid * 128, 128)
            for k in range(n_slabs):
                pltpu.async_copy(
                    tab_t_hbm.at[pl.ds(8 * k, 8), pl.ds(c0, 128)],
                    bufs.at[slot, k], semd.at[slot])

        def wait_column(slot):
            pltpu.make_async_copy(
                tab_t_hbm.at[pl.ds(0, 64), pl.ds(0, 128)],
                bufs.at[slot], semd.at[slot]).wait()

        carry = (0, 0, 0)
        for seg in range(n_segs):
            lo = seg_bounds[seg]
            hi = seg_bounds[seg + 1]
            nt_seg = jnp.clip(n_own - seg * 32, 0, 32)

            @pl.when(nt_seg > 0)
            def _prime(_seg=seg):
                fire_column(_seg * 32, 0)

            def chunk_body(tp, car, _seg=seg, _lo=lo, _hi=hi, _nt=nt_seg):
                t = _seg * 32 + tp
                dslot = tp & 1

                @pl.when(tp + 1 < _nt)
                def _next():
                    fire_column(t + 1, 1 - dslot)

                wait_column(dslot)
                cid = wid + _NW * t
                nc = rescan(cid, _lo, _hi)
                return extract_groups(nc, cid * 128, dslot, car)

            carry = lax.fori_loop(0, nt_seg, chunk_body, carry,
                                  unroll=False)

        def _tail(car):
            for k in range(n_slabs):
                pltpu.async_copy(
                    tail_t_hbm.at[pl.ds(8 * k, 8)], bufs.at[0, k],
                    semd.at[0])
            wait_column(0)
            lo = seg_bounds[n_segs - 1]
            hi = seg_bounds[n_segs]
            nc = rescan(n_tc, lo, hi)
            return extract_groups(nc, n_tc * 128, 0, car)

        carry = lax.cond(wid == tail_owner, _tail, lambda c: c, carry)
        fill, slot, nfired = carry

        lax.cond(fill > 0, lambda: _fire(slot), lambda: None)
        nfired = nfired + jnp.where(fill > 0, 1, 0)

        def _drain_body(i, c):
            _drain_one()
            return c

        lax.fori_loop(0, jnp.minimum(nfired, 2), _drain_body, 0,
                      unroll=False)

    return lookup_kernel


def kernel(labels, train, table):
    original_shape = labels.shape
    flat = labels.reshape(-1).astype(jnp.int32)
    key = jax.random.key(42)
    drop_ids = jax.random.uniform(key, flat.shape) < _DROPOUT_PROB
    train_on = jnp.asarray(train) != 0
    flat = jnp.where(
        jnp.logical_and(train_on, drop_ids),
        jnp.full_like(flat, _NUM_CLASSES),
        flat,
    )
    b = flat.shape[0]
    d = table.shape[1]
    n_tc = table.shape[0] // 128
    tail_t = jnp.pad(table[n_tc * 128:, :].T,
                     ((0, 0), (0, 128 - (table.shape[0] - n_tc * 128))))
    out_raw = _make_lookup(table.shape[0], d, b)(flat, table.T, tail_t)
    return out_raw[:b, :d].reshape(*original_shape, -1)


# speedup vs baseline: 8.2393x; 8.2393x over previous
import functools

import jax
import jax.numpy as jnp
from jax import lax
from jax.experimental import pallas as pl
from jax.experimental.pallas import tpu as pltpu
from jax.experimental.pallas import tpu_sc as plsc

_NUM_CLASSES = 1000000
_DROPOUT_PROB = 0.1

_NC = 2
_NS = 16
_NW = _NC * _NS
_CHUNK = 128


@functools.lru_cache(maxsize=None)
def _make_gather(vocab_pad: int, b: int):
    b_per_w = b // _NW
    n_chunks = b_per_w // _CHUNK
    mesh = plsc.VectorSubcoreMesh(core_axis_name="c", subcore_axis_name="s")

    @functools.partial(
        pl.kernel,
        out_type=jax.ShapeDtypeStruct((b, 128), jnp.float32),
        mesh=mesh,
        scratch_types=[
            pltpu.VMEM((n_chunks, _CHUNK), jnp.int32),
            pltpu.VMEM((b_per_w, 128), jnp.float32),
            pltpu.SemaphoreType.DMA,
        ],
    )
    def gather_kernel(idx_hbm, table_hbm, out_hbm, idx_v, rows_v, sem):
        wid = lax.axis_index("s") * _NC + lax.axis_index("c")
        base = wid * b_per_w
        for j in range(n_chunks):
            pltpu.sync_copy(idx_hbm.at[pl.ds(base + j * _CHUNK, _CHUNK)],
                            idx_v.at[j])
        copies = [
            pltpu.async_copy(
                table_hbm.at[idx_v.at[j]],
                rows_v.at[pl.ds(j * _CHUNK, _CHUNK)],
                sem,
            )
            for j in range(n_chunks)
        ]
        for cp in copies:
            cp.wait()
        pltpu.sync_copy(rows_v, out_hbm.at[pl.ds(base, b_per_w)])

    return gather_kernel


def kernel(labels, train, table):
    original_shape = labels.shape
    flat = labels.reshape(-1).astype(jnp.int32)
    key = jax.random.key(42)
    drop_ids = jax.random.uniform(key, flat.shape) < _DROPOUT_PROB
    train_on = jnp.asarray(train) != 0
    flat = jnp.where(
        jnp.logical_and(train_on, drop_ids),
        jnp.full_like(flat, _NUM_CLASSES),
        flat,
    )
    b = flat.shape[0]
    d = table.shape[1]
    v = table.shape[0]
    v_pad = ((v + 7) // 8) * 8
    tbl128 = jnp.pad(table, ((0, v_pad - v), (0, 128 - d)))
    out_raw = _make_gather(v_pad, b)(flat, tbl128)
    return out_raw[:, :d].reshape(*original_shape, -1)


# speedup vs baseline: 11.3523x; 1.3778x over previous
import functools

import jax
import jax.numpy as jnp
from jax import lax
from jax.experimental import pallas as pl
from jax.experimental.pallas import tpu as pltpu
from jax.experimental.pallas import tpu_sc as plsc

_NUM_CLASSES = 1000000
_DROPOUT_PROB = 0.1
_NC = 2
_NS = 16
_NW = _NC * _NS
_BATCH = 64


@functools.lru_cache(maxsize=None)
def _make_gather(vocab: int, d: int, b: int):
    b_per_w = b // _NW
    n_batches = b_per_w // _BATCH
    mesh = plsc.VectorSubcoreMesh(core_axis_name="c", subcore_axis_name="s")

    @functools.partial(
        pl.kernel,
        out_type=jax.ShapeDtypeStruct((b, 128), jnp.float32),
        mesh=mesh,
        scratch_types=[
            pltpu.VMEM((b_per_w,), jnp.int32),
            pltpu.VMEM((_BATCH, 8, d), jnp.float32),
            pltpu.VMEM((_BATCH, 128), jnp.float32),
            pltpu.SemaphoreType.DMA,
        ],
        compiler_params=pltpu.CompilerParams(needs_layout_passes=False),
    )
    def gather_kernel(idx_hbm, table_hbm, out_hbm, idx_v, bufs, stag, sem):
        wid = lax.axis_index("s") * _NC + lax.axis_index("c")
        base = wid * b_per_w
        pltpu.sync_copy(idx_hbm.at[pl.ds(base, b_per_w)], idx_v)
        iota = lax.iota(jnp.int32, 16)
        for g in range(n_batches):
            ivals = []
            for v in range(_BATCH // 16):
                i_vec = idx_v[pl.ds(g * _BATCH + v * 16, 16)]
                for lane in range(16):
                    ivals.append(jnp.sum(jnp.where(iota == lane, i_vec, 0)))
            for t in range(_BATCH):
                i8 = pl.multiple_of((ivals[t] >> 3) * 8, 8)
                pltpu.async_copy(table_hbm.at[pl.ds(i8, 8)], bufs.at[t], sem)
            pltpu.make_async_copy(
                table_hbm.at[pl.ds(0, 8)], bufs, sem).wait()
            for t in range(_BATCH):
                r = ivals[t] & 7
                for v in range(d // 16):
                    stag[t, pl.ds(16 * v, 16)] = bufs[t, r,
                                                      pl.ds(16 * v, 16)]
            pltpu.sync_copy(stag, out_hbm.at[pl.ds(base + g * _BATCH,
                                                   _BATCH)])

    return gather_kernel


def kernel(labels, train, table):
    original_shape = labels.shape
    flat = labels.reshape(-1).astype(jnp.int32)
    key = jax.random.key(42)
    drop_ids = jax.random.uniform(key, flat.shape) < _DROPOUT_PROB
    train_on = jnp.asarray(train) != 0
    flat = jnp.where(
        jnp.logical_and(train_on, drop_ids),
        jnp.full_like(flat, _NUM_CLASSES),
        flat,
    )
    b = flat.shape[0]
    d = table.shape[1]
    out_raw = _make_gather(table.shape[0], d, b)(flat, table)
    return out_raw[:, :d].reshape(*original_shape, -1)
